# baseline (speedup 1.0000x reference)
import functools

import jax
import jax.numpy as jnp
from jax import lax
from jax.experimental import pallas as pl
from jax.experimental.pallas import tpu as pltpu
from jax.experimental.pallas import tpu_sc as plsc

N = 10000
E = 320000
D = 128
H = 64
NP = 10240
L = 4

NCORE = 2
NSUB = 16
EPT = E // NSUB
K = 80
NIT = EPT // K
RPT = NP // NSUB

_MESH = plsc.VectorSubcoreMesh(core_axis_name="c", subcore_axis_name="s")


def _deg_body(src_hbm, dst_hbm, ones_hbm, z_hbm, deg_s_hbm, deg_d_hbm,
              s0, d0, s1, d1, ones_v, deg_s_sh, deg_d_sh,
              is0, id0, is1, id1):
    c = lax.axis_index("c")
    s = lax.axis_index("s")
    pltpu.sync_copy(ones_hbm, ones_v)
    pltpu.sync_copy(z_hbm.at[pl.ds(s * RPT, RPT)],
                    deg_s_sh.at[pl.ds(s * RPT, RPT)])
    pltpu.sync_copy(z_hbm.at[pl.ds(s * RPT, RPT)],
                    deg_d_sh.at[pl.ds(s * RPT, RPT)])
    plsc.subcore_barrier()
    base = c * EPC + s * EPT2

    pltpu.async_copy(src_hbm.at[pl.ds(base, K)], s0, is0)
    pltpu.async_copy(dst_hbm.at[pl.ds(base, K)], d0, id0)

    @pl.loop(0, (NIT2 - 1) // 2)
    def _(j):
        i0 = 2 * j
        pltpu.async_copy(src_hbm.at[pl.ds(base + (i0 + 1) * K, K)], s1, is1)
        pltpu.async_copy(dst_hbm.at[pl.ds(base + (i0 + 1) * K, K)], d1, id1)
        pltpu.make_async_copy(src_hbm.at[pl.ds(base, K)], s0, is0).wait()
        pltpu.make_async_copy(dst_hbm.at[pl.ds(base, K)], d0, id0).wait()
        a = pltpu.async_copy(ones_v, deg_s_sh.at[s0], is0, add=True)
        b = pltpu.async_copy(ones_v, deg_d_sh.at[d0], id0, add=True)
        a.wait()
        b.wait()
        pltpu.async_copy(src_hbm.at[pl.ds(base + (i0 + 2) * K, K)], s0, is0)
        pltpu.async_copy(dst_hbm.at[pl.ds(base + (i0 + 2) * K, K)], d0, id0)
        pltpu.make_async_copy(src_hbm.at[pl.ds(base, K)], s1, is1).wait()
        pltpu.make_async_copy(dst_hbm.at[pl.ds(base, K)], d1, id1).wait()
        a2 = pltpu.async_copy(ones_v, deg_s_sh.at[s1], is1, add=True)
        b2 = pltpu.async_copy(ones_v, deg_d_sh.at[d1], id1, add=True)
        a2.wait()
        b2.wait()

    pltpu.make_async_copy(src_hbm.at[pl.ds(base, K)], s0, is0).wait()
    pltpu.make_async_copy(dst_hbm.at[pl.ds(base, K)], d0, id0).wait()
    a = pltpu.async_copy(ones_v, deg_s_sh.at[s0], is0, add=True)
    b = pltpu.async_copy(ones_v, deg_d_sh.at[d0], id0, add=True)
    a.wait()
    b.wait()

    plsc.subcore_barrier()
    pltpu.sync_copy(deg_s_sh.at[pl.ds(s * RPT, RPT)],
                    deg_s_hbm.at[pl.ds(c * NP + s * RPT, RPT)])
    pltpu.sync_copy(deg_d_sh.at[pl.ds(s * RPT, RPT)],
                    deg_d_hbm.at[pl.ds(c * NP + s * RPT, RPT)])


_deg_call = pl.kernel(
    _deg_body,
    out_type=(jax.ShapeDtypeStruct((NCORE * NP,), jnp.float32),
              jax.ShapeDtypeStruct((NCORE * NP,), jnp.float32)),
    mesh=_MESH,
    scratch_types=[
        pltpu.VMEM((K,), jnp.int32),
        pltpu.VMEM((K,), jnp.int32),
        pltpu.VMEM((K,), jnp.int32),
        pltpu.VMEM((K,), jnp.int32),
        pltpu.VMEM((K,), jnp.float32),
        pltpu.VMEM_SHARED((NP,), jnp.float32),
        pltpu.VMEM_SHARED((NP,), jnp.float32),
        pltpu.SemaphoreType.DMA,
        pltpu.SemaphoreType.DMA,
        pltpu.SemaphoreType.DMA,
        pltpu.SemaphoreType.DMA,
    ],
)


EPC = E // NCORE
EPT2 = EPC // NSUB
NIT2 = EPT2 // K


KA = 64
NFULL = EPT2 // KA
RK = EPT2 - NFULL * KA
JLOOP = (NFULL - 3) // 3


def _agg_body(m_hbm, src_hbm, dst_hbm, z_hbm, agg_hbm,
              sidx, d0, d1, d2, dr, r0, r1, r2, rr, agg_sh,
              sg0, sg1, sg2, si0, si1, si2, ss0, ss1, ss2, sr):
    c = lax.axis_index("c")
    s = lax.axis_index("s")
    base = c * EPC + s * EPT2
    pltpu.sync_copy(src_hbm.at[pl.ds(base, EPT2)], sidx)

    def fetch(t, d, r, si, sg):
        pltpu.async_copy(dst_hbm.at[pl.ds(base + t * KA, KA)], d, si)
        pltpu.async_copy(m_hbm.at[sidx.at[pl.ds(t * KA, KA)]], r, sg)

    def wait_fetch(d, r, si, sg):
        pltpu.make_async_copy(dst_hbm.at[pl.ds(base, KA)], d, si).wait()
        pltpu.make_async_copy(m_hbm.at[pl.ds(0, KA)], r, sg).wait()

    def wait_scatter(d, r, ss):
        pltpu.make_async_copy(r, agg_sh.at[d], ss).wait()

    fetch(0, d0, r0, si0, sg0)
    fetch(1, d1, r1, si1, sg1)
    fetch(2, d2, r2, si2, sg2)
    pltpu.sync_copy(z_hbm.at[pl.ds(s * RPT, RPT)],
                    agg_sh.at[pl.ds(s * RPT, RPT)])
    plsc.subcore_barrier()

    @pl.loop(0, JLOOP)
    def _(j):
        t = 3 * j
        wait_fetch(d0, r0, si0, sg0)
        pltpu.async_copy(r0, agg_sh.at[d0], ss0, add=True)
        wait_fetch(d1, r1, si1, sg1)
        pltpu.async_copy(r1, agg_sh.at[d1], ss1, add=True)
        wait_scatter(d0, r0, ss0)
        fetch(t + 3, d0, r0, si0, sg0)
        wait_fetch(d2, r2, si2, sg2)
        pltpu.async_copy(r2, agg_sh.at[d2], ss2, add=True)
        wait_scatter(d1, r1, ss1)
        fetch(t + 4, d1, r1, si1, sg1)
        wait_scatter(d2, r2, ss2)
        fetch(t + 5, d2, r2, si2, sg2)

    def drain_sync(d, r, si, sg):
        wait_fetch(d, r, si, sg)
        pltpu.sync_copy(r, agg_sh.at[d], add=True)

    off = base + NFULL * KA
    pltpu.async_copy(dst_hbm.at[pl.ds(off, RK)], dr, sr)
    pltpu.async_copy(m_hbm.at[sidx.at[pl.ds(NFULL * KA, RK)]], rr, sr)
    drain_sync(d0, r0, si0, sg0)
    drain_sync(d1, r1, si1, sg1)
    drain_sync(d2, r2, si2, sg2)
    pltpu.make_async_copy(dst_hbm.at[pl.ds(base, RK)], dr, sr).wait()
    pltpu.make_async_copy(m_hbm.at[pl.ds(0, RK)], rr, sr).wait()
    pltpu.sync_copy(rr, agg_sh.at[dr], add=True)

    plsc.subcore_barrier()
    pltpu.sync_copy(agg_sh.at[pl.ds(s * RPT, RPT)],
                    agg_hbm.at[c, pl.ds(s * RPT, RPT)])


_agg_call = pl.kernel(
    _agg_body,
    out_type=jax.ShapeDtypeStruct((NCORE, NP, D), jnp.float32),
    mesh=_MESH,
    scratch_types=[
        pltpu.VMEM((EPT2,), jnp.int32),
        pltpu.VMEM((KA,), jnp.int32),
        pltpu.VMEM((KA,), jnp.int32),
        pltpu.VMEM((KA,), jnp.int32),
        pltpu.VMEM((RK,), jnp.int32),
        pltpu.VMEM((KA, D), jnp.float32),
        pltpu.VMEM((KA, D), jnp.float32),
        pltpu.VMEM((KA, D), jnp.float32),
        pltpu.VMEM((RK, D), jnp.float32),
        pltpu.VMEM_SHARED((NP, D), jnp.float32),
        pltpu.SemaphoreType.DMA,
        pltpu.SemaphoreType.DMA,
        pltpu.SemaphoreType.DMA,
        pltpu.SemaphoreType.DMA,
        pltpu.SemaphoreType.DMA,
        pltpu.SemaphoreType.DMA,
        pltpu.SemaphoreType.DMA,
        pltpu.SemaphoreType.DMA,
        pltpu.SemaphoreType.DMA,
        pltpu.SemaphoreType.DMA,
    ],
)


R = 2000
G = N // R


def _norm(d):
    return jnp.where(d > 0, lax.rsqrt(jnp.maximum(d, 1.0)), 0.0)


def _prologue_body(h_ref, w_ref, b_ref, dsrc_ref, h0_ref, m_ref):
    h0 = jnp.dot(h_ref[...], w_ref[...],
                 preferred_element_type=jnp.float32) + b_ref[...]
    h0_ref[...] = h0
    ns = _norm(dsrc_ref[0, :, 0] + dsrc_ref[1, :, 0])
    m_ref[...] = h0 * ns[:, None]


_prologue_call = pl.pallas_call(
    _prologue_body,
    grid=(G,),
    in_specs=[
        pl.BlockSpec((R, D), lambda i: (i, 0)),
        pl.BlockSpec((D, D), lambda i: (0, 0)),
        pl.BlockSpec((1, D), lambda i: (0, 0)),
        pl.BlockSpec((NCORE, R, 1), lambda i: (0, i, 0)),
    ],
    out_specs=[
        pl.BlockSpec((R, D), lambda i: (i, 0)),
        pl.BlockSpec((R, D), lambda i: (i, 0)),
    ],
    out_shape=[
        jax.ShapeDtypeStruct((N, D), jnp.float32),
        jax.ShapeDtypeStruct((N, D), jnp.float32),
    ],
)


def _layer_body(is_last, agg_ref, h_ref, w_ref, b_ref, g_ref, bt_ref,
                dsrc_ref, ddst_ref, *outs):
    nd = _norm(ddst_ref[0, :, 0] + ddst_ref[1, :, 0])
    a = (agg_ref[0] + agg_ref[1]) * nd[:, None]
    hl = (jnp.dot(a, w_ref[...], preferred_element_type=jnp.float32)
          + b_ref[...])
    hl = jnp.maximum(g_ref[...] * hl + bt_ref[...], 0.0)
    hout = h_ref[...] + hl
    if is_last:
        hg_ref, = outs

        @pl.when(pl.program_id(0) == 0)
        def _():
            hg_ref[...] = jnp.zeros_like(hg_ref)

        hg_ref[...] += jnp.sum(hout, axis=0, keepdims=True) * (1.0 / N)
    else:
        hout_ref, m_ref = outs
        hout_ref[...] = hout
        ns = _norm(dsrc_ref[0, :, 0] + dsrc_ref[1, :, 0])
        m_ref[...] = hout * ns[:, None]


def _make_layer(is_last):
    if is_last:
        out_shape = [jax.ShapeDtypeStruct((1, D), jnp.float32)]
        out_specs = [pl.BlockSpec((1, D), lambda i: (0, 0))]
    else:
        out_shape = [
            jax.ShapeDtypeStruct((N, D), jnp.float32),
            jax.ShapeDtypeStruct((N, D), jnp.float32),
        ]
        out_specs = [
            pl.BlockSpec((R, D), lambda i: (i, 0)),
            pl.BlockSpec((R, D), lambda i: (i, 0)),
        ]
    return pl.pallas_call(
        functools.partial(_layer_body, is_last),
        grid=(G,),
        in_specs=[
            pl.BlockSpec((NCORE, R, D), lambda i: (0, i, 0)),
            pl.BlockSpec((R, D), lambda i: (i, 0)),
            pl.BlockSpec((D, D), lambda i: (0, 0)),
            pl.BlockSpec((1, D), lambda i: (0, 0)),
            pl.BlockSpec((1, D), lambda i: (0, 0)),
            pl.BlockSpec((1, D), lambda i: (0, 0)),
            pl.BlockSpec((NCORE, R, 1), lambda i: (0, i, 0)),
            pl.BlockSpec((NCORE, R, 1), lambda i: (0, i, 0)),
        ],
        out_specs=out_specs,
        out_shape=out_shape,
    )


_layer_mid = _make_layer(False)
_layer_last = _make_layer(True)


def kernel(h, edge_index, e, W_embed, b_embed, Ws, bs, gammas, betas):
    del e
    ei = edge_index.astype(jnp.int32)
    src = ei[0]
    dst = ei[1]
    ones_v = jnp.ones((K,), jnp.float32)
    zeros = jnp.zeros((NP, D), jnp.float32)
    z1 = jnp.zeros((NP,), jnp.float32)
    deg_s, deg_d = _deg_call(src, dst, ones_v, z1)
    deg_s = deg_s.reshape(NCORE, NP, 1)
    deg_d = deg_d.reshape(NCORE, NP, 1)
    hcur, m = _prologue_call(h, W_embed, b_embed.reshape(1, D), deg_s)
    for l in range(L):
        agg = _agg_call(m, src, dst, zeros)
        b2 = bs[l].reshape(1, D)
        g2 = gammas[l].reshape(1, D)
        bt2 = betas[l].reshape(1, D)
        if l < L - 1:
            hcur, m = _layer_mid(agg, hcur, Ws[l], b2, g2, bt2, deg_s,
                                 deg_d)
        else:
            hg, = _layer_last(agg, hcur, Ws[l], b2, g2, bt2, deg_s, deg_d)
    return hg


# speedup vs baseline: 1.1489x; 1.1489x over previous
import functools

import jax
import jax.numpy as jnp
from jax import lax
from jax.experimental import pallas as pl
from jax.experimental.pallas import tpu as pltpu
from jax.experimental.pallas import tpu_sc as plsc

N = 10000
E = 320000
D = 128
H = 64
NP = 10240
L = 4

NCORE = 2
NSUB = 16
RPT = NP // NSUB
EPC = E // NCORE
EPT2 = EPC // NSUB

_MESH = plsc.VectorSubcoreMesh(core_axis_name="c", subcore_axis_name="s")


KD = 128
NFD = EPT2 // KD
RKD = EPT2 - NFD * KD


def _deg_body(src_hbm, dst_hbm, ones_hbm, z_hbm, deg_s_hbm, deg_d_hbm,
              s0, d0, s1, d1, sr_, dr_, ones_v, deg_s_sh, deg_d_sh,
              is0, id0, is1, id1):
    c = lax.axis_index("c")
    s = lax.axis_index("s")
    pltpu.sync_copy(ones_hbm, ones_v)
    pltpu.sync_copy(z_hbm.at[pl.ds(s * RPT, RPT)],
                    deg_s_sh.at[pl.ds(s * RPT, RPT)])
    pltpu.sync_copy(z_hbm.at[pl.ds(s * RPT, RPT)],
                    deg_d_sh.at[pl.ds(s * RPT, RPT)])
    plsc.subcore_barrier()
    base = c * EPC + s * EPT2

    pltpu.async_copy(src_hbm.at[pl.ds(base, KD)], s0, is0)
    pltpu.async_copy(dst_hbm.at[pl.ds(base, KD)], d0, id0)

    @pl.loop(0, NFD // 2 - 1)
    def _(j):
        i0 = 2 * j
        pltpu.async_copy(src_hbm.at[pl.ds(base + (i0 + 1) * KD, KD)], s1,
                         is1)
        pltpu.async_copy(dst_hbm.at[pl.ds(base + (i0 + 1) * KD, KD)], d1,
                         id1)
        pltpu.make_async_copy(src_hbm.at[pl.ds(base, KD)], s0, is0).wait()
        pltpu.make_async_copy(dst_hbm.at[pl.ds(base, KD)], d0, id0).wait()
        a = pltpu.async_copy(ones_v, deg_s_sh.at[s0], is0, add=True)
        b = pltpu.async_copy(ones_v, deg_d_sh.at[d0], id0, add=True)
        a.wait()
        b.wait()
        pltpu.async_copy(src_hbm.at[pl.ds(base + (i0 + 2) * KD, KD)], s0,
                         is0)
        pltpu.async_copy(dst_hbm.at[pl.ds(base + (i0 + 2) * KD, KD)], d0,
                         id0)
        pltpu.make_async_copy(src_hbm.at[pl.ds(base, KD)], s1, is1).wait()
        pltpu.make_async_copy(dst_hbm.at[pl.ds(base, KD)], d1, id1).wait()
        a2 = pltpu.async_copy(ones_v, deg_s_sh.at[s1], is1, add=True)
        b2 = pltpu.async_copy(ones_v, deg_d_sh.at[d1], id1, add=True)
        a2.wait()
        b2.wait()

    pltpu.async_copy(src_hbm.at[pl.ds(base + (NFD - 1) * KD, KD)], s1, is1)
    pltpu.async_copy(dst_hbm.at[pl.ds(base + (NFD - 1) * KD, KD)], d1, id1)
    pltpu.make_async_copy(src_hbm.at[pl.ds(base, KD)], s0, is0).wait()
    pltpu.make_async_copy(dst_hbm.at[pl.ds(base, KD)], d0, id0).wait()
    a = pltpu.async_copy(ones_v, deg_s_sh.at[s0], is0, add=True)
    b = pltpu.async_copy(ones_v, deg_d_sh.at[d0], id0, add=True)
    a.wait()
    b.wait()
    pltpu.async_copy(src_hbm.at[pl.ds(base + NFD * KD, RKD)], sr_, is0)
    pltpu.async_copy(dst_hbm.at[pl.ds(base + NFD * KD, RKD)], dr_, id0)
    pltpu.make_async_copy(src_hbm.at[pl.ds(base, KD)], s1, is1).wait()
    pltpu.make_async_copy(dst_hbm.at[pl.ds(base, KD)], d1, id1).wait()
    a2 = pltpu.async_copy(ones_v, deg_s_sh.at[s1], is1, add=True)
    b2 = pltpu.async_copy(ones_v, deg_d_sh.at[d1], id1, add=True)
    a2.wait()
    b2.wait()
    pltpu.make_async_copy(src_hbm.at[pl.ds(base, RKD)], sr_, is0).wait()
    pltpu.make_async_copy(dst_hbm.at[pl.ds(base, RKD)], dr_, id0).wait()
    a3 = pltpu.async_copy(ones_v.at[pl.ds(0, RKD)], deg_s_sh.at[sr_], is0,
                          add=True)
    b3 = pltpu.async_copy(ones_v.at[pl.ds(0, RKD)], deg_d_sh.at[dr_], id0,
                          add=True)
    a3.wait()
    b3.wait()

    plsc.subcore_barrier()
    pltpu.sync_copy(deg_s_sh.at[pl.ds(s * RPT, RPT)],
                    deg_s_hbm.at[pl.ds(c * NP + s * RPT, RPT)])
    pltpu.sync_copy(deg_d_sh.at[pl.ds(s * RPT, RPT)],
                    deg_d_hbm.at[pl.ds(c * NP + s * RPT, RPT)])


_deg_call = pl.kernel(
    _deg_body,
    out_type=(jax.ShapeDtypeStruct((NCORE * NP,), jnp.float32),
              jax.ShapeDtypeStruct((NCORE * NP,), jnp.float32)),
    mesh=_MESH,
    scratch_types=[
        pltpu.VMEM((KD,), jnp.int32),
        pltpu.VMEM((KD,), jnp.int32),
        pltpu.VMEM((KD,), jnp.int32),
        pltpu.VMEM((KD,), jnp.int32),
        pltpu.VMEM((RKD,), jnp.int32),
        pltpu.VMEM((RKD,), jnp.int32),
        pltpu.VMEM((KD,), jnp.float32),
        pltpu.VMEM_SHARED((NP,), jnp.float32),
        pltpu.VMEM_SHARED((NP,), jnp.float32),
        pltpu.SemaphoreType.DMA,
        pltpu.SemaphoreType.DMA,
        pltpu.SemaphoreType.DMA,
        pltpu.SemaphoreType.DMA,
    ],
)


KA = 128
NFULL = EPT2 // KA
RK = EPT2 - NFULL * KA


def _agg_body(m_hbm, src_hbm, dst_hbm, z_hbm, agg_hbm,
              sidx, d0, d1, dr, r0, r1, rr, agg_sh,
              sg0, sg1, si0, si1, sr):
    c = lax.axis_index("c")
    s = lax.axis_index("s")
    base = c * EPC + s * EPT2
    pltpu.sync_copy(src_hbm.at[pl.ds(base, EPT2)], sidx)

    def fetch(t, d, r, si, sg):
        pltpu.async_copy(dst_hbm.at[pl.ds(base + t * KA, KA)], d, si)
        pltpu.async_copy(m_hbm.at[sidx.at[pl.ds(t * KA, KA)]], r, sg)

    def drain_scatter(d, r, si, sg):
        pltpu.make_async_copy(m_hbm.at[pl.ds(0, KA)], r, sg).wait()
        pltpu.make_async_copy(dst_hbm.at[pl.ds(base, KA)], d, si).wait()
        pltpu.sync_copy(r, agg_sh.at[d], add=True)

    fetch(0, d0, r0, si0, sg0)
    pltpu.sync_copy(z_hbm.at[pl.ds(s * RPT, RPT)],
                    agg_sh.at[pl.ds(s * RPT, RPT)])
    plsc.subcore_barrier()

    @pl.loop(0, NFULL // 2 - 1)
    def _(j):
        i0 = 2 * j
        fetch(i0 + 1, d1, r1, si1, sg1)
        drain_scatter(d0, r0, si0, sg0)
        fetch(i0 + 2, d0, r0, si0, sg0)
        drain_scatter(d1, r1, si1, sg1)

    fetch(NFULL - 1, d1, r1, si1, sg1)
    drain_scatter(d0, r0, si0, sg0)
    pltpu.async_copy(dst_hbm.at[pl.ds(base + NFULL * KA, RK)], dr, sr)
    pltpu.async_copy(m_hbm.at[sidx.at[pl.ds(NFULL * KA, RK)]], rr, sr)
    drain_scatter(d1, r1, si1, sg1)
    pltpu.make_async_copy(dst_hbm.at[pl.ds(base, RK)], dr, sr).wait()
    pltpu.make_async_copy(m_hbm.at[pl.ds(0, RK)], rr, sr).wait()
    pltpu.sync_copy(rr, agg_sh.at[dr], add=True)

    plsc.subcore_barrier()
    pltpu.sync_copy(agg_sh.at[pl.ds(s * RPT, RPT)],
                    agg_hbm.at[c, pl.ds(s * RPT, RPT)])


_agg_call = pl.kernel(
    _agg_body,
    out_type=jax.ShapeDtypeStruct((NCORE, NP, D), jnp.float32),
    mesh=_MESH,
    scratch_types=[
        pltpu.VMEM((EPT2,), jnp.int32),
        pltpu.VMEM((KA,), jnp.int32),
        pltpu.VMEM((KA,), jnp.int32),
        pltpu.VMEM((RK,), jnp.int32),
        pltpu.VMEM((KA, D), jnp.float32),
        pltpu.VMEM((KA, D), jnp.float32),
        pltpu.VMEM((RK, D), jnp.float32),
        pltpu.VMEM_SHARED((NP, D), jnp.float32),
        pltpu.SemaphoreType.DMA,
        pltpu.SemaphoreType.DMA,
        pltpu.SemaphoreType.DMA,
        pltpu.SemaphoreType.DMA,
        pltpu.SemaphoreType.DMA,
    ],
)


R = 2000
G = N // R


def _norm(d):
    return jnp.where(d > 0, lax.rsqrt(jnp.maximum(d, 1.0)), 0.0)


def _prologue_body(h_ref, w_ref, b_ref, dsrc_ref, h0_ref, m_ref):
    h0 = jnp.dot(h_ref[...], w_ref[...],
                 preferred_element_type=jnp.float32) + b_ref[...]
    h0_ref[...] = h0
    ns = _norm(dsrc_ref[0, :, 0] + dsrc_ref[1, :, 0])
    m_ref[...] = h0 * ns[:, None]


_prologue_call = pl.pallas_call(
    _prologue_body,
    grid=(G,),
    in_specs=[
        pl.BlockSpec((R, D), lambda i: (i, 0)),
        pl.BlockSpec((D, D), lambda i: (0, 0)),
        pl.BlockSpec((1, D), lambda i: (0, 0)),
        pl.BlockSpec((NCORE, R, 1), lambda i: (0, i, 0)),
    ],
    out_specs=[
        pl.BlockSpec((R, D), lambda i: (i, 0)),
        pl.BlockSpec((R, D), lambda i: (i, 0)),
    ],
    out_shape=[
        jax.ShapeDtypeStruct((N, D), jnp.float32),
        jax.ShapeDtypeStruct((N, D), jnp.float32),
    ],
)


def _layer_body(is_last, agg_ref, h_ref, w_ref, b_ref, g_ref, bt_ref,
                dsrc_ref, ddst_ref, *outs):
    nd = _norm(ddst_ref[0, :, 0] + ddst_ref[1, :, 0])
    a = (agg_ref[0] + agg_ref[1]) * nd[:, None]
    hl = (jnp.dot(a, w_ref[...], preferred_element_type=jnp.float32)
          + b_ref[...])
    hl = jnp.maximum(g_ref[...] * hl + bt_ref[...], 0.0)
    hout = h_ref[...] + hl
    if is_last:
        hg_ref, = outs

        @pl.when(pl.program_id(0) == 0)
        def _():
            hg_ref[...] = jnp.zeros_like(hg_ref)

        hg_ref[...] += jnp.sum(hout, axis=0, keepdims=True) * (1.0 / N)
    else:
        hout_ref, m_ref = outs
        hout_ref[...] = hout
        ns = _norm(dsrc_ref[0, :, 0] + dsrc_ref[1, :, 0])
        m_ref[...] = hout * ns[:, None]


def _make_layer(is_last):
    if is_last:
        out_shape = [jax.ShapeDtypeStruct((1, D), jnp.float32)]
        out_specs = [pl.BlockSpec((1, D), lambda i: (0, 0))]
    else:
        out_shape = [
            jax.ShapeDtypeStruct((N, D), jnp.float32),
            jax.ShapeDtypeStruct((N, D), jnp.float32),
        ]
        out_specs = [
            pl.BlockSpec((R, D), lambda i: (i, 0)),
            pl.BlockSpec((R, D), lambda i: (i, 0)),
        ]
    return pl.pallas_call(
        functools.partial(_layer_body, is_last),
        grid=(G,),
        in_specs=[
            pl.BlockSpec((NCORE, R, D), lambda i: (0, i, 0)),
            pl.BlockSpec((R, D), lambda i: (i, 0)),
            pl.BlockSpec((D, D), lambda i: (0, 0)),
            pl.BlockSpec((1, D), lambda i: (0, 0)),
            pl.BlockSpec((1, D), lambda i: (0, 0)),
            pl.BlockSpec((1, D), lambda i: (0, 0)),
            pl.BlockSpec((NCORE, R, 1), lambda i: (0, i, 0)),
            pl.BlockSpec((NCORE, R, 1), lambda i: (0, i, 0)),
        ],
        out_specs=out_specs,
        out_shape=out_shape,
    )


_layer_mid = _make_layer(False)
_layer_last = _make_layer(True)


def kernel(h, edge_index, e, W_embed, b_embed, Ws, bs, gammas, betas):
    del e
    ei = edge_index.astype(jnp.int32)
    src = ei[0]
    dst = ei[1]
    ones_v = jnp.ones((KD,), jnp.float32)
    zeros = jnp.zeros((NP, D), jnp.float32)
    z1 = jnp.zeros((NP,), jnp.float32)
    deg_s, deg_d = _deg_call(src, dst, ones_v, z1)
    deg_s = deg_s.reshape(NCORE, NP, 1)
    deg_d = deg_d.reshape(NCORE, NP, 1)
    hcur, m = _prologue_call(h, W_embed, b_embed.reshape(1, D), deg_s)
    for l in range(L):
        agg = _agg_call(m, src, dst, zeros)
        b2 = bs[l].reshape(1, D)
        g2 = gammas[l].reshape(1, D)
        bt2 = betas[l].reshape(1, D)
        if l < L - 1:
            hcur, m = _layer_mid(agg, hcur, Ws[l], b2, g2, bt2, deg_s,
                                 deg_d)
        else:
            hg, = _layer_last(agg, hcur, Ws[l], b2, g2, bt2, deg_s, deg_d)
    return hg
